# Initial kernel scaffold; baseline (speedup 1.0000x reference)
import functools

import jax
import jax.numpy as jnp
from jax import lax
from jax.experimental import pallas as pl
from jax.experimental.pallas import tpu as pltpu
from jax.experimental.pallas import tpu_sc as plsc

_K = 16
_M = 8
_P = 16
_LP = 4
_D = 64
_N, _H, _W = 8, 32, 32
_HW = _H * _W
_PIX = _N * _HW
_J = _M * _P
_TABLE_ROWS = _M * (2 ** _K)

_NC, _NS, _L = 2, 16, 16
_NWORKERS = _NC * _NS
_PPW = _PIX // _NWORKERS


def _votes_kernel(b_ref, idx_ref, at_ref):
    for m in range(_M):
        T = [b_ref[:, m * _K + k, :] for k in range(_K)]
        base = T[0] * 0.0 + 1.0
        wb = jnp.zeros((_N, _HW), jnp.int32)
        ba = []
        for k in range(_K):
            base = base * jnp.maximum(T[k], 1.0 - T[k])
            wb = wb + jnp.where(T[k] >= 0.5, jnp.int32(1 << k), jnp.int32(0))
            ba.append(jnp.abs(T[k] - 0.5))
        abi = []
        aba = []
        for _ in range(_LP):
            minval = ba[0]
            minidx = jnp.zeros((_N, _HW), jnp.int32)
            mval = T[0]
            for k in range(1, _K):
                cmp = ba[k] < minval
                minval = jnp.where(cmp, ba[k], minval)
                minidx = jnp.where(cmp, jnp.int32(k), minidx)
                mval = jnp.where(cmp, T[k], mval)
            abi.append(minidx)
            aba.append(mval)
            ba = [jnp.where(minidx == k, ba[k] + 1.0, ba[k])
                  for k in range(_K)]
        denom = jnp.maximum(aba[0], 1.0 - aba[0])
        for j in range(1, _LP):
            denom = denom * jnp.maximum(aba[j], 1.0 - aba[j])
        scale = base / denom
        bw = [jnp.left_shift(jnp.int32(1), abi[j]) for j in range(_LP)]
        wb_clear = wb
        for j in range(_LP):
            wb_clear = wb_clear - (
                jnp.bitwise_and(jnp.right_shift(wb, abi[j]), 1) * bw[j])
        for p in range(_P):
            it = wb_clear + jnp.int32(m * (2 ** _K))
            fac = scale
            for l in range(_LP):
                if (p >> l) & 1:
                    it = it + bw[l]
                    fac = fac * aba[l]
                else:
                    fac = fac * (1.0 - aba[l])
            idx_ref[m * _P + p, :, :] = it
            at_ref[m * _P + p, :, :] = fac


def _dispatch_kernel(table, idx_hbm, at_hbm, out_hbm,
                     idx_v, at_v, rows0, rows1, out_v, sem0, sem1):
    wid = lax.axis_index("s") * _NC + lax.axis_index("c")
    base = wid * _PPW

    pltpu.sync_copy(idx_hbm.at[pl.ds(base, _PPW), :], idx_v)
    pltpu.sync_copy(at_hbm.at[pl.ds(base, _PPW), :], at_v)

    bufs = (rows0, rows1)
    sems = (sem0, sem1)

    def start(p, b):
        pltpu.async_copy(table.at[idx_v.at[p]], bufs[b], sems[b])

    def wait(p, b):
        pltpu.make_async_copy(table.at[idx_v.at[p]], bufs[b], sems[b]).wait()

    def accum(p, b):
        rows = bufs[b]

        def jbody(j, accs):
            a = at_v[p, j]
            return tuple(
                accs[c] + a * rows[j, pl.ds(c * _L, _L)] for c in range(4))

        zeros = tuple(jnp.zeros((_L,), jnp.float32) for _ in range(4))
        accs = lax.fori_loop(0, _J, jbody, zeros, unroll=8)
        for c in range(4):
            out_v[p, pl.ds(c * _L, _L)] = accs[c]

    start(0, 0)

    def pair_body(i, _):
        p0 = 2 * i
        start(p0 + 1, 1)
        wait(p0, 0)
        accum(p0, 0)

        @pl.when(i < _PPW // 2 - 1)
        def _():
            start(p0 + 2, 0)

        wait(p0 + 1, 1)
        accum(p0 + 1, 1)
        return 0

    lax.fori_loop(0, _PPW // 2, pair_body, 0)
    pltpu.sync_copy(out_v, out_hbm.at[pl.ds(base, _PPW), :])


@jax.jit
def kernel(B, weights, bias):
    b3 = B.reshape(_N, _M * _K, _HW)
    idx, at = pl.pallas_call(
        _votes_kernel,
        out_shape=(
            jax.ShapeDtypeStruct((_J, _N, _HW), jnp.int32),
            jax.ShapeDtypeStruct((_J, _N, _HW), jnp.float32),
        ),
    )(b3)

    idx_t = idx.reshape(_J, _PIX).T
    at_t = at.reshape(_J, _PIX).T
    wflat = weights.reshape(_TABLE_ROWS, _D)

    mesh = plsc.VectorSubcoreMesh(core_axis_name="c", subcore_axis_name="s")
    acc = pl.kernel(
        _dispatch_kernel,
        out_type=jax.ShapeDtypeStruct((_PIX, _D), jnp.float32),
        mesh=mesh,
        scratch_types=[
            pltpu.VMEM((_PPW, _J), jnp.int32),
            pltpu.VMEM((_PPW, _J), jnp.float32),
            pltpu.VMEM((_J, _D), jnp.float32),
            pltpu.VMEM((_J, _D), jnp.float32),
            pltpu.VMEM((_PPW, _D), jnp.float32),
            pltpu.SemaphoreType.DMA,
            pltpu.SemaphoreType.DMA,
        ],
    )(wflat, idx_t, at_t)

    out = acc.reshape(_N, _H, _W, _D) + bias
    return jnp.transpose(out, (0, 3, 1, 2))


# speedup vs baseline: 9.9944x; 9.9944x over previous
import functools

import jax
import jax.numpy as jnp
from jax import lax
from jax.experimental import pallas as pl
from jax.experimental.pallas import tpu as pltpu
from jax.experimental.pallas import tpu_sc as plsc

_K = 16
_M = 8
_P = 16
_LP = 4
_D = 64
_N, _H, _W = 8, 32, 32
_HW = _H * _W
_PIX = _N * _HW
_J = _M * _P
_TABLE_ROWS = _M * (2 ** _K)

_NC, _NS, _L = 2, 16, 16
_NWORKERS = _NC * _NS
_PPW = _PIX // _NWORKERS


def _votes_kernel(b_ref, idx_ref, at_ref):
    for m in range(_M):
        T = [b_ref[:, m * _K + k, :] for k in range(_K)]
        base = T[0] * 0.0 + 1.0
        wb = jnp.zeros((_N, _HW), jnp.int32)
        ba = []
        for k in range(_K):
            base = base * jnp.maximum(T[k], 1.0 - T[k])
            wb = wb + jnp.where(T[k] >= 0.5, jnp.int32(1 << k), jnp.int32(0))
            ba.append(jnp.abs(T[k] - 0.5))
        abi = []
        aba = []
        for _ in range(_LP):
            minval = ba[0]
            minidx = jnp.zeros((_N, _HW), jnp.int32)
            mval = T[0]
            for k in range(1, _K):
                cmp = ba[k] < minval
                minval = jnp.where(cmp, ba[k], minval)
                minidx = jnp.where(cmp, jnp.int32(k), minidx)
                mval = jnp.where(cmp, T[k], mval)
            abi.append(minidx)
            aba.append(mval)
            ba = [jnp.where(minidx == k, ba[k] + 1.0, ba[k])
                  for k in range(_K)]
        denom = jnp.maximum(aba[0], 1.0 - aba[0])
        for j in range(1, _LP):
            denom = denom * jnp.maximum(aba[j], 1.0 - aba[j])
        scale = base / denom
        bw = [jnp.left_shift(jnp.int32(1), abi[j]) for j in range(_LP)]
        wb_clear = wb
        for j in range(_LP):
            wb_clear = wb_clear - (
                jnp.bitwise_and(jnp.right_shift(wb, abi[j]), 1) * bw[j])
        for p in range(_P):
            it = wb_clear + jnp.int32(m * (2 ** _K))
            fac = scale
            for l in range(_LP):
                if (p >> l) & 1:
                    it = it + bw[l]
                    fac = fac * aba[l]
                else:
                    fac = fac * (1.0 - aba[l])
            idx_ref[m * _P + p, :, :] = it
            at_ref[m * _P + p, :, :] = fac


def _dispatch_kernel(table, idx_hbm, at_hbm, out_hbm,
                     idx_v, at_v, rows0, rows1, out_v, sem0, sem1):
    wid = lax.axis_index("s") * _NC + lax.axis_index("c")
    base = wid * _PPW

    pltpu.sync_copy(idx_hbm.at[pl.ds(base, _PPW), :], idx_v)
    pltpu.sync_copy(at_hbm.at[pl.ds(base, _PPW), :], at_v)

    bufs = (rows0, rows1)
    sems = (sem0, sem1)

    def start(p, b):
        pltpu.async_copy(table.at[idx_v.at[p]], bufs[b], sems[b])

    def wait(p, b):
        pltpu.make_async_copy(table.at[idx_v.at[p]], bufs[b], sems[b]).wait()

    def accum(p, b):
        rows = bufs[b]

        def jbody(jb, accs):
            av = at_v[p, pl.ds(jb * _L, _L)]
            for q in range(_L):
                a = av[q]
                j = jb * _L + q
                accs = tuple(
                    accs[c] + a * rows[j, pl.ds(c * _L, _L)]
                    for c in range(4))
            return accs

        zeros = tuple(jnp.zeros((_L,), jnp.float32) for _ in range(4))
        accs = lax.fori_loop(0, _J // _L, jbody, zeros)
        for c in range(4):
            out_v[p, pl.ds(c * _L, _L)] = accs[c]

    start(0, 0)

    def pair_body(i, _):
        p0 = 2 * i
        start(p0 + 1, 1)
        wait(p0, 0)
        accum(p0, 0)

        @pl.when(i < _PPW // 2 - 1)
        def _():
            start(p0 + 2, 0)

        wait(p0 + 1, 1)
        accum(p0 + 1, 1)
        return 0

    lax.fori_loop(0, _PPW // 2, pair_body, 0)
    pltpu.sync_copy(out_v, out_hbm.at[pl.ds(base, _PPW), :])


@jax.jit
def kernel(B, weights, bias):
    b3 = B.reshape(_N, _M * _K, _HW)
    idx, at = pl.pallas_call(
        _votes_kernel,
        out_shape=(
            jax.ShapeDtypeStruct((_J, _N, _HW), jnp.int32),
            jax.ShapeDtypeStruct((_J, _N, _HW), jnp.float32),
        ),
    )(b3)

    idx_t = idx.reshape(_J, _PIX).T
    at_t = at.reshape(_J, _PIX).T
    wflat = weights.reshape(_TABLE_ROWS, _D)

    mesh = plsc.VectorSubcoreMesh(core_axis_name="c", subcore_axis_name="s")
    acc = pl.kernel(
        _dispatch_kernel,
        out_type=jax.ShapeDtypeStruct((_PIX, _D), jnp.float32),
        mesh=mesh,
        scratch_types=[
            pltpu.VMEM((_PPW, _J), jnp.int32),
            pltpu.VMEM((_PPW, _J), jnp.float32),
            pltpu.VMEM((_J, _D), jnp.float32),
            pltpu.VMEM((_J, _D), jnp.float32),
            pltpu.VMEM((_PPW, _D), jnp.float32),
            pltpu.SemaphoreType.DMA,
            pltpu.SemaphoreType.DMA,
        ],
        compiler_params=pltpu.CompilerParams(use_tc_tiling_on_sc=False),
    )(wflat, idx_t, at_t)

    out = acc.reshape(_N, _H, _W, _D) + bias
    return jnp.transpose(out, (0, 3, 1, 2))


# speedup vs baseline: 10.6618x; 1.0668x over previous
import jax
import jax.numpy as jnp
from jax import lax
from jax.experimental import pallas as pl
from jax.experimental.pallas import tpu as pltpu
from jax.experimental.pallas import tpu_sc as plsc

_K = 16
_M = 8
_P = 16
_LP = 4
_D = 64
_N, _H, _W = 8, 32, 32
_HW = _H * _W
_PIX = _N * _HW
_J = _M * _P
_TABLE_ROWS = _M * (2 ** _K)

_NC, _NS, _L = 2, 16, 16
_NWORKERS = _NC * _NS
_PPW = _PIX // _NWORKERS
_NCH = _PPW // _L
_NBUF = 4


def _fern_kernel(b_hbm, table, out_hbm,
                 b_v, idx_v, at_v, rows0, rows1, rows2, rows3, out_v,
                 sem0, sem1, sem2, sem3):
    wid = lax.axis_index("s") * _NC + lax.axis_index("c")
    base = wid * _PPW
    n = base // _HW
    hw0 = base % _HW

    lanes = lax.broadcasted_iota(jnp.int32, (_L,), 0)

    def fern_body(m, _):
        pltpu.sync_copy(
            b_hbm.at[n, pl.ds(m * _K, _K), pl.ds(hw0, _PPW)], b_v)

        def chunk_body(pc, _):
            cols = pl.ds(pc * _L, _L)
            T = [b_v[k, cols] for k in range(_K)]
            base_p = jnp.maximum(T[0], 1.0 - T[0])
            wb = jnp.where(T[0] >= 0.5, jnp.int32(1), jnp.int32(0))
            ba = [jnp.abs(T[0] - 0.5)]
            for k in range(1, _K):
                base_p = base_p * jnp.maximum(T[k], 1.0 - T[k])
                wb = wb + jnp.where(T[k] >= 0.5,
                                    jnp.int32(1 << k), jnp.int32(0))
                ba.append(jnp.abs(T[k] - 0.5))
            abi = []
            aba = []
            for _j in range(_LP):
                minval = ba[0]
                minidx = jnp.zeros((_L,), jnp.int32)
                mval = T[0]
                for k in range(1, _K):
                    cmp = ba[k] < minval
                    minval = jnp.where(cmp, ba[k], minval)
                    minidx = jnp.where(cmp, jnp.int32(k), minidx)
                    mval = jnp.where(cmp, T[k], mval)
                abi.append(minidx)
                aba.append(mval)
                if _j < _LP - 1:
                    ba = [jnp.where(minidx == k, ba[k] + 1.0, ba[k])
                          for k in range(_K)]
            denom = jnp.maximum(aba[0], 1.0 - aba[0])
            for j in range(1, _LP):
                denom = denom * jnp.maximum(aba[j], 1.0 - aba[j])
            scale = base_p / denom
            bw = [jnp.left_shift(jnp.int32(1), abi[j]) for j in range(_LP)]
            wb_clear = wb + m * (2 ** _K)
            for j in range(_LP):
                wb_clear = wb_clear - (
                    jnp.bitwise_and(jnp.right_shift(wb, abi[j]), 1) * bw[j])
            omaba = [1.0 - aba[j] for j in range(_LP)]
            dst = (pc * _L + lanes) * _J + m * _P
            for p in range(_P):
                it = wb_clear
                fac = scale
                for l in range(_LP):
                    if (p >> l) & 1:
                        it = it + bw[l]
                        fac = fac * aba[l]
                    else:
                        fac = fac * omaba[l]
                plsc.store_scatter(idx_v, [dst + p], it)
                plsc.store_scatter(at_v, [dst + p], fac)
            return 0

        lax.fori_loop(0, _NCH, chunk_body, 0)
        return 0

    lax.fori_loop(0, _M, fern_body, 0)

    bufs = (rows0, rows1, rows2, rows3)
    sems = (sem0, sem1, sem2, sem3)

    def start(p, b):
        pltpu.async_copy(
            table.at[idx_v.at[pl.ds(p * _J, _J)]], bufs[b], sems[b])

    def wait(p, b):
        pltpu.make_async_copy(
            table.at[idx_v.at[pl.ds(p * _J, _J)]], bufs[b], sems[b]).wait()

    def accum(p, b):
        rows = bufs[b]

        def jbody(jb, accs):
            av = at_v[pl.ds(p * _J + jb * _L, _L)]
            for q in range(_L):
                a = av[q]
                j = jb * _L + q
                accs = tuple(
                    accs[c] + a * rows[j, pl.ds(c * _L, _L)]
                    for c in range(4))
            return accs

        zeros = tuple(jnp.zeros((_L,), jnp.float32) for _ in range(4))
        accs = lax.fori_loop(0, _J // _L, jbody, zeros)
        for c in range(4):
            out_v[p, pl.ds(c * _L, _L)] = accs[c]

    for p in range(_NBUF - 1):
        start(p, p)

    def ring_body(ib, _):
        p0 = ib * _NBUF
        for r in range(_NBUF):
            p = p0 + r

            @pl.when(p + _NBUF - 1 < _PPW)
            def _():
                start(p + _NBUF - 1, (r + _NBUF - 1) % _NBUF)

            wait(p, r)
            accum(p, r)
        return 0

    lax.fori_loop(0, _PPW // _NBUF, ring_body, 0)
    pltpu.sync_copy(out_v, out_hbm.at[pl.ds(base, _PPW), :])


@jax.jit
def kernel(B, weights, bias):
    b3 = B.reshape(_N, _M * _K, _HW)
    wflat = weights.reshape(_TABLE_ROWS, _D)

    mesh = plsc.VectorSubcoreMesh(core_axis_name="c", subcore_axis_name="s")
    acc = pl.kernel(
        _fern_kernel,
        out_type=jax.ShapeDtypeStruct((_PIX, _D), jnp.float32),
        mesh=mesh,
        scratch_types=[
            pltpu.VMEM((_K, _PPW), jnp.float32),
            pltpu.VMEM((_PPW * _J,), jnp.int32),
            pltpu.VMEM((_PPW * _J,), jnp.float32),
            pltpu.VMEM((_J, _D), jnp.float32),
            pltpu.VMEM((_J, _D), jnp.float32),
            pltpu.VMEM((_J, _D), jnp.float32),
            pltpu.VMEM((_J, _D), jnp.float32),
            pltpu.VMEM((_PPW, _D), jnp.float32),
            pltpu.SemaphoreType.DMA,
            pltpu.SemaphoreType.DMA,
            pltpu.SemaphoreType.DMA,
            pltpu.SemaphoreType.DMA,
        ],
        compiler_params=pltpu.CompilerParams(
            use_tc_tiling_on_sc=False, needs_layout_passes=False),
    )(b3, wflat)

    out = acc.reshape(_N, _H, _W, _D) + bias
    return jnp.transpose(out, (0, 3, 1, 2))


# speedup vs baseline: 11.8029x; 1.1070x over previous
import jax
import jax.numpy as jnp
from jax import lax
from jax.experimental import pallas as pl
from jax.experimental.pallas import tpu as pltpu
from jax.experimental.pallas import tpu_sc as plsc

_K = 16
_M = 8
_P = 16
_LP = 4
_D = 64
_N, _H, _W = 8, 32, 32
_HW = _H * _W
_PIX = _N * _HW
_J = _M * _P
_TABLE_ROWS = _M * (2 ** _K)

_NC, _NS, _L = 2, 16, 16
_NWORKERS = _NC * _NS
_PPW = _PIX // _NWORKERS
_NCH = _PPW // _L
_NBUF = 4


def _votes_kernel(b_hbm, idx_hbm, at_hbm, b_v, idx_v, at_v):
    wid = lax.axis_index("s") * _NC + lax.axis_index("c")
    base = wid * _PPW
    n = base // _HW
    hw0 = base % _HW

    lanes = lax.broadcasted_iota(jnp.int32, (_L,), 0)

    def fern_body(m, _):
        pltpu.sync_copy(
            b_hbm.at[n, pl.ds(m * _K, _K), pl.ds(hw0, _PPW)], b_v)

        def chunk_body(pc, _):
            cols = pl.ds(pc * _L, _L)
            T = [b_v[k, cols] for k in range(_K)]
            base_p = jnp.maximum(T[0], 1.0 - T[0])
            wb = jnp.where(T[0] >= 0.5, jnp.int32(1), jnp.int32(0))
            ba = [jnp.abs(T[0] - 0.5)]
            for k in range(1, _K):
                base_p = base_p * jnp.maximum(T[k], 1.0 - T[k])
                wb = wb + jnp.where(T[k] >= 0.5,
                                    jnp.int32(1 << k), jnp.int32(0))
                ba.append(jnp.abs(T[k] - 0.5))
            abi = []
            aba = []
            for _j in range(_LP):
                minval = ba[0]
                minidx = jnp.zeros((_L,), jnp.int32)
                mval = T[0]
                for k in range(1, _K):
                    cmp = ba[k] < minval
                    minval = jnp.where(cmp, ba[k], minval)
                    minidx = jnp.where(cmp, jnp.int32(k), minidx)
                    mval = jnp.where(cmp, T[k], mval)
                abi.append(minidx)
                aba.append(mval)
                if _j < _LP - 1:
                    ba = [jnp.where(minidx == k, ba[k] + 1.0, ba[k])
                          for k in range(_K)]
            denom = jnp.maximum(aba[0], 1.0 - aba[0])
            for j in range(1, _LP):
                denom = denom * jnp.maximum(aba[j], 1.0 - aba[j])
            scale = base_p / denom
            bw = [jnp.left_shift(jnp.int32(1), abi[j]) for j in range(_LP)]
            wb_clear = wb + m * (2 ** _K)
            for j in range(_LP):
                wb_clear = wb_clear - (
                    jnp.bitwise_and(jnp.right_shift(wb, abi[j]), 1) * bw[j])
            omaba = [1.0 - aba[j] for j in range(_LP)]
            pix = pc * _L + lanes
            dst = pix * _J + m * _P
            for p in range(_P):
                it = wb_clear
                fac = scale
                for l in range(_LP):
                    if (p >> l) & 1:
                        it = it + bw[l]
                        fac = fac * aba[l]
                    else:
                        fac = fac * omaba[l]
                plsc.store_scatter(idx_v, [dst + p], it)
                plsc.store_scatter(at_v, [dst + p], fac)
            return 0

        lax.fori_loop(0, _NCH, chunk_body, 0)
        return 0

    lax.fori_loop(0, _M, fern_body, 0)
    pltpu.sync_copy(idx_v, idx_hbm.at[pl.ds(base * _J, _PPW * _J)])
    pltpu.sync_copy(at_v, at_hbm.at[pl.ds(base * _J, _PPW * _J)])


def _gather_kernel(table, idx_hbm, at_hbm, out_hbm,
                   idx_v, at_v, rows0, rows1, rows2, rows3, out_v,
                   sem0, sem1, sem2, sem3):
    wid = lax.axis_index("s") * _NC + lax.axis_index("c")
    base = wid * _PPW
    pltpu.sync_copy(idx_hbm.at[pl.ds(base * _J, _PPW * _J)], idx_v)
    pltpu.sync_copy(at_hbm.at[pl.ds(base * _J, _PPW * _J)], at_v)

    bufs = (rows0, rows1, rows2, rows3)
    sems = (sem0, sem1, sem2, sem3)

    def start(p, b):
        pltpu.async_copy(
            table.at[idx_v.at[pl.ds(p * _J, _J)]], bufs[b], sems[b])

    def wait(p, b):
        pltpu.make_async_copy(
            table.at[idx_v.at[pl.ds(p * _J, _J)]], bufs[b], sems[b]).wait()

    def accum(p, b):
        rows = bufs[b]

        def jbody(jb, accs):
            av = at_v[pl.ds(p * _J + jb * _L, _L)]
            for q in range(_L):
                a = av[q]
                j = jb * _L + q
                accs = tuple(
                    accs[c] + a * rows[j, pl.ds(c * _L, _L)]
                    for c in range(4))
            return accs

        zeros = tuple(jnp.zeros((_L,), jnp.float32) for _ in range(4))
        accs = lax.fori_loop(0, _J // _L, jbody, zeros)
        for c in range(4):
            out_v[p, pl.ds(c * _L, _L)] = accs[c]

    for p in range(_NBUF - 1):
        start(p, p)

    def ring_body(ib, _):
        p0 = ib * _NBUF
        for r in range(_NBUF):
            p = p0 + r

            @pl.when(p + _NBUF - 1 < _PPW)
            def _():
                start(p + _NBUF - 1, (r + _NBUF - 1) % _NBUF)

            wait(p, r)
            accum(p, r)
        return 0

    lax.fori_loop(0, _PPW // _NBUF, ring_body, 0)
    pltpu.sync_copy(out_v, out_hbm.at[pl.ds(base, _PPW), :])


@jax.jit
def kernel(B, weights, bias):
    b3 = B.reshape(_N, _M * _K, _HW)

    mesh = plsc.VectorSubcoreMesh(core_axis_name="c", subcore_axis_name="s")

    wflat = weights.reshape(_TABLE_ROWS, _D)

    idx, at = pl.kernel(
        _votes_kernel,
        out_type=(
            jax.ShapeDtypeStruct((_PIX * _J,), jnp.int32),
            jax.ShapeDtypeStruct((_PIX * _J,), jnp.float32),
        ),
        mesh=mesh,
        scratch_types=[
            pltpu.VMEM((_K, _PPW), jnp.float32),
            pltpu.VMEM((_PPW * _J,), jnp.int32),
            pltpu.VMEM((_PPW * _J,), jnp.float32),
        ],
        compiler_params=pltpu.CompilerParams(
            use_tc_tiling_on_sc=False, needs_layout_passes=False),
    )(b3)

    acc = pl.kernel(
        _gather_kernel,
        out_type=jax.ShapeDtypeStruct((_PIX, _D), jnp.float32),
        mesh=mesh,
        scratch_types=[
            pltpu.VMEM((_PPW * _J,), jnp.int32),
            pltpu.VMEM((_PPW * _J,), jnp.float32),
            pltpu.VMEM((_J, _D), jnp.float32),
            pltpu.VMEM((_J, _D), jnp.float32),
            pltpu.VMEM((_J, _D), jnp.float32),
            pltpu.VMEM((_J, _D), jnp.float32),
            pltpu.VMEM((_PPW, _D), jnp.float32),
            pltpu.SemaphoreType.DMA,
            pltpu.SemaphoreType.DMA,
            pltpu.SemaphoreType.DMA,
            pltpu.SemaphoreType.DMA,
        ],
        compiler_params=pltpu.CompilerParams(
            use_tc_tiling_on_sc=False, needs_layout_passes=False),
    )(wflat, idx, at)

    out = acc.reshape(_N, _H, _W, _D) + bias
    return jnp.transpose(out, (0, 3, 1, 2))
